# Initial kernel scaffold; baseline (speedup 1.0000x reference)
import jax
import jax.numpy as jnp
from jax.experimental import pallas as pl


def kernel(x, scale, gate_w, gate_b, mlp1_weight, mlp1_bias, mlp2_weight, mlp2_bias):
    raise NotImplementedError("write your pallas kernel here")


# speedup vs baseline: 3.7489x; 3.7489x over previous
import jax
import jax.numpy as jnp
from jax.experimental import pallas as pl
from jax.experimental.pallas import tpu as pltpu


def _moe_body(F, x_ref, scale_ref, gate_w_ref, gate_b_ref,
              w1_ref, b1_ref, w2_ref, b2_ref,
              out_ref, t_scr, w_scr):
    e = pl.program_id(0)

    @pl.when(e == 0)
    def _routing():
        x = x_ref[...]
        t = x * jax.lax.rsqrt(jnp.mean(x * x, axis=-1, keepdims=True) + 1e-5)
        t = t * scale_ref[...]
        t_scr[...] = t
        g = jax.lax.dot_general(
            t, gate_w_ref[...], (((1,), (1,)), ((), ())),
            preferred_element_type=jnp.float32) + gate_b_ref[...]
        ncols = g.shape[-1]
        col = jax.lax.broadcasted_iota(jnp.int32, g.shape, 1)
        v1 = jnp.max(g, axis=-1, keepdims=True)
        e1 = jnp.min(jnp.where(g == v1, col, ncols), axis=-1, keepdims=True)
        first1 = (col == e1)
        g2 = jnp.where(first1, -1e30, g)
        v2 = jnp.max(g2, axis=-1, keepdims=True)
        e2 = jnp.min(jnp.where(g2 == v2, col, ncols), axis=-1, keepdims=True)
        first2 = (col == e2)
        p1 = jax.nn.sigmoid(v1 - v2)
        p2 = 1.0 - p1
        w_scr[...] = jnp.where(first1, p1, 0.0) + jnp.where(first2, p2, 0.0)
        out_ref[...] = x

    t = t_scr[...]
    h = jax.lax.dot_general(
        t, w1_ref[0], (((1,), (1,)), ((), ())),
        preferred_element_type=jnp.float32) + b1_ref[0]
    x_glu = h[:, :F]
    x_lin = h[:, F:]
    a = x_glu * jax.nn.sigmoid(1.702 * x_glu) * (x_lin + 1.0)
    o = jax.lax.dot_general(
        a, w2_ref[0], (((1,), (1,)), ((), ())),
        preferred_element_type=jnp.float32) + b2_ref[0]
    w_all = w_scr[...]
    ecol = jax.lax.broadcasted_iota(jnp.int32, w_all.shape, 1)
    wcol = jnp.sum(jnp.where(ecol == e, w_all, 0.0), axis=1, keepdims=True)
    out_ref[...] += o * wcol


def kernel(x, scale, gate_w, gate_b, mlp1_weight, mlp1_bias, mlp2_weight, mlp2_bias):
    B, D = x.shape
    E, twoF, _ = mlp1_weight.shape
    F = twoF // 2

    scale2 = scale.reshape(1, D)
    gate_b2 = gate_b.reshape(1, E)
    b1_3d = mlp1_bias.reshape(E, 1, twoF)
    b2_3d = mlp2_bias.reshape(E, 1, D)

    grid = (E,)
    out = pl.pallas_call(
        lambda *refs: _moe_body(F, *refs),
        grid=grid,
        in_specs=[
            pl.BlockSpec((B, D), lambda e: (0, 0)),
            pl.BlockSpec((1, D), lambda e: (0, 0)),
            pl.BlockSpec((E, D), lambda e: (0, 0)),
            pl.BlockSpec((1, E), lambda e: (0, 0)),
            pl.BlockSpec((1, twoF, D), lambda e: (e, 0, 0)),
            pl.BlockSpec((1, 1, twoF), lambda e: (e, 0, 0)),
            pl.BlockSpec((1, D, F), lambda e: (e, 0, 0)),
            pl.BlockSpec((1, 1, D), lambda e: (e, 0, 0)),
        ],
        out_specs=pl.BlockSpec((B, D), lambda e: (0, 0)),
        out_shape=jax.ShapeDtypeStruct((B, D), jnp.float32),
        scratch_shapes=[
            pltpu.VMEM((B, D), jnp.float32),
            pltpu.VMEM((B, E), jnp.float32),
        ],
        compiler_params=pltpu.CompilerParams(
            dimension_semantics=("arbitrary",),
        ),
    )(x, scale2, gate_w, gate_b2, mlp1_weight, b1_3d, mlp2_weight, b2_3d)
    return out
